# baseline (speedup 1.0000x reference)
import functools
import math

import jax
import jax.numpy as jnp
import numpy as np
from jax import lax
from jax.experimental import pallas as pl
from jax.experimental.pallas import tpu as pltpu
from jax.experimental.pallas import tpu_sc as plsc

EMB = 1024
HALF = EMB // 2
NC = 2
NS = 16
NW = NC * NS
CHUNK = 32


@functools.lru_cache(maxsize=None)
def _angle_factors(seq_len):
    scale = math.log(10000) / (HALF - 1)
    freqs = np.exp(np.arange(HALF, dtype=np.float64) * -scale)
    a = np.arange(0, seq_len, 64, dtype=np.float64)
    b = np.arange(1, 65, dtype=np.float64)
    sa = np.sin(a[:, None] * freqs[None, :])
    ca = np.cos(a[:, None] * freqs[None, :])
    sb = np.sin(b[:, None] * freqs[None, :])
    cb = np.cos(b[:, None] * freqs[None, :])
    f32 = lambda m: m.astype(np.float32)
    sa2 = f32(np.concatenate([sa, sa], axis=1))
    ca2 = f32(np.concatenate([ca, ca], axis=1))
    xx = f32(np.concatenate([cb, -sb], axis=1))
    yy = f32(np.concatenate([sb, cb], axis=1))
    return sa2, ca2, xx, yy


def _pos_table(seq_len):
    sa2, ca2, xx, yy = (jnp.asarray(m) for m in _angle_factors(seq_len))
    tab = (sa2[:, None, :] * xx[None, :, :]
           + ca2[:, None, :] * yy[None, :, :])
    return tab.reshape(seq_len, EMB)


@functools.lru_cache(maxsize=None)
def _row0_rep():
    row0 = np.concatenate(
        [np.zeros((HALF,), np.float32), np.ones((HALF,), np.float32)])
    return np.tile(row0[None, :], (16, 1))


def _make_sc_kernel(batch, seq_len):
    rows_per_w = seq_len // NW
    n_chunks = rows_per_w // CHUNK
    mesh = plsc.VectorSubcoreMesh(core_axis_name="c", subcore_axis_name="s")

    n_pairs = n_chunks // 2

    def body(x_hbm, table_hbm, row0_hbm, out_hbm, x_v, row0_v, buf0, buf1,
             sg0, sg1, ss0, ss1, sem_aux):
        cid = lax.axis_index("c")
        sid = lax.axis_index("s")
        w = sid * NC + cid
        base = w * rows_per_w

        def g_desc(ci, buf, sem):
            return pltpu.make_async_copy(
                table_hbm.at[pl.ds(base + ci * CHUNK, CHUNK)], buf, sem)

        def s_desc(ci, b, buf, sem):
            return pltpu.make_async_copy(
                buf, out_hbm.at[b, pl.ds(base + ci * CHUNK, CHUNK)], sem)

        def aux_descs():
            descs = [pltpu.make_async_copy(row0_hbm, row0_v, sem_aux)]
            for b in range(batch):
                descs.append(pltpu.make_async_copy(
                    x_hbm.at[b, pl.ds(base, rows_per_w)], x_v.at[b],
                    sem_aux))
            return descs

        def fixups(ci):
            s0 = base + ci * CHUNK
            for b in range(batch):
                for j in range(CHUNK // 16):
                    xvec = x_v[b, pl.ds(ci * CHUNK + j * 16, 16)]
                    m = xvec == 0
                    npad = plsc.all_reduce_population_count(m)[0]

                    @pl.when(npad > 0)
                    def _fix():
                        g0 = s0 + j * 16
                        p = g0 + lax.iota(jnp.int32, 16)
                        first = plsc.all_reduce_ffs(m)
                        idx = jnp.where(m, p, g0 + first)
                        pltpu.async_copy(row0_v, out_hbm.at[b].at[idx],
                                         sem_aux).wait()

        for d in aux_descs():
            d.start()

        def d_desc(b, sem):
            return pltpu.make_async_copy(
                table_hbm.at[pl.ds(base, rows_per_w)],
                out_hbm.at[b, pl.ds(base, rows_per_w)], sem)

        for b in range(batch):
            d_desc(b, (sg0, sg1, ss0, ss1)[b]).start()
        for d in aux_descs():
            d.wait()
        for b in range(batch):
            d_desc(b, (sg0, sg1, ss0, ss1)[b]).wait()

        def chunk_step(ci, carry):
            fixups(ci)
            return carry

        lax.fori_loop(0, n_chunks, chunk_step, 0)

    return pl.kernel(
        body,
        mesh=mesh,
        compiler_params=pltpu.CompilerParams(needs_layout_passes=False),
        out_type=jax.ShapeDtypeStruct((batch, seq_len, EMB), jnp.float32),
        scratch_types=[
            pltpu.VMEM((batch, rows_per_w), jnp.int32),
            pltpu.VMEM((16, EMB), jnp.float32),
            pltpu.VMEM((CHUNK, EMB), jnp.float32),
            pltpu.VMEM((CHUNK, EMB), jnp.float32),
            pltpu.SemaphoreType.DMA,
            pltpu.SemaphoreType.DMA,
            pltpu.SemaphoreType.DMA,
            pltpu.SemaphoreType.DMA,
            pltpu.SemaphoreType.DMA,
        ],
    )


def kernel(x):
    batch, seq_len = x.shape
    table = _pos_table(seq_len)
    row0 = _row0_rep()
    return _make_sc_kernel(batch, seq_len)(x, table, row0)


# speedup vs baseline: 45.0466x; 45.0466x over previous
import functools
import math

import jax
import jax.numpy as jnp
import numpy as np
from jax import lax
from jax.experimental import pallas as pl
from jax.experimental.pallas import tpu as pltpu
from jax.experimental.pallas import tpu_sc as plsc

EMB = 1024
HALF = EMB // 2
NC = 2
NS = 16
NW = NC * NS
CHUNK = 64


@functools.lru_cache(maxsize=None)
def _angle_factors(seq_len):
    scale = math.log(10000) / (HALF - 1)
    freqs = np.exp(np.arange(HALF, dtype=np.float64) * -scale)
    a = np.arange(0, seq_len, 64, dtype=np.float64)
    b = np.arange(1, 65, dtype=np.float64)
    sa = np.sin(a[:, None] * freqs[None, :])
    ca = np.cos(a[:, None] * freqs[None, :])
    sb = np.sin(b[:, None] * freqs[None, :])
    cb = np.cos(b[:, None] * freqs[None, :])
    f32 = lambda m: m.astype(np.float32)
    sa2 = f32(np.concatenate([sa, sa], axis=1))
    ca2 = f32(np.concatenate([ca, ca], axis=1))
    xx = f32(np.concatenate([cb, -sb], axis=1))
    yy = f32(np.concatenate([sb, cb], axis=1))
    return sa2, ca2, xx, yy


def _pos_table(seq_len):
    sa2, ca2, xx, yy = (jnp.asarray(m) for m in _angle_factors(seq_len))
    tab = (sa2[:, None, :] * xx[None, :, :]
           + ca2[:, None, :] * yy[None, :, :])
    return tab.reshape(seq_len, EMB)


@functools.lru_cache(maxsize=None)
def _row0_rep():
    row0 = np.concatenate(
        [np.zeros((HALF,), np.float32), np.ones((HALF,), np.float32)])
    return np.tile(row0[None, :], (16, 1))


def _make_sc_kernel(batch, seq_len):
    rows_per_w = seq_len // NW
    n_chunks = rows_per_w // CHUNK
    mesh = plsc.VectorSubcoreMesh(core_axis_name="c", subcore_axis_name="s")

    def body(x_hbm, table_hbm, row0_hbm, out_hbm, x_v, row0_v, buf0,
             sg0, ss0, sem_aux):
        cid = lax.axis_index("c")
        sid = lax.axis_index("s")
        w = sid * NC + cid
        base = w * rows_per_w

        def g_desc(ci, buf, sem):
            return pltpu.make_async_copy(
                table_hbm.at[pl.ds(base + ci * CHUNK, CHUNK)], buf, sem)

        def s_desc(ci, b, buf, sem):
            return pltpu.make_async_copy(
                buf, out_hbm.at[b, pl.ds(base + ci * CHUNK, CHUNK)], sem)

        def aux_descs():
            descs = [pltpu.make_async_copy(row0_hbm, row0_v, sem_aux)]
            for b in range(batch):
                descs.append(pltpu.make_async_copy(
                    x_hbm.at[b, pl.ds(base, rows_per_w)], x_v.at[b],
                    sem_aux))
            return descs

        def fixups(ci):
            s0 = base + ci * CHUNK
            for b in range(batch):
                for j in range(CHUNK // 16):
                    xvec = x_v[b, pl.ds(ci * CHUNK + j * 16, 16)]
                    m = xvec == 0
                    npad = plsc.all_reduce_population_count(m)[0]

                    @pl.when(npad > 0)
                    def _fix():
                        g0 = s0 + j * 16
                        p = g0 + lax.iota(jnp.int32, 16)
                        first = plsc.all_reduce_ffs(m)
                        idx = jnp.where(m, p, g0 + first)
                        pltpu.async_copy(row0_v, out_hbm.at[b].at[idx],
                                         sem_aux).wait()

        for d in aux_descs():
            d.start()
        g_desc(0, buf0, sg0).start()
        for d in aux_descs():
            d.wait()

        for ci in range(n_chunks):
            g_desc(ci, buf0, sg0).wait()
            for b in range(batch):
                s_desc(ci, b, buf0, ss0).start()
            for b in range(batch):
                s_desc(ci, b, buf0, ss0).wait()
            if ci + 1 < n_chunks:
                g_desc(ci + 1, buf0, sg0).start()

        def fix_step(ci, carry):
            fixups(ci)
            return carry

        lax.fori_loop(0, n_chunks, fix_step, 0)

    return pl.kernel(
        body,
        mesh=mesh,
        compiler_params=pltpu.CompilerParams(needs_layout_passes=False),
        out_type=jax.ShapeDtypeStruct((batch, seq_len, EMB), jnp.float32),
        scratch_types=[
            pltpu.VMEM((batch, rows_per_w), jnp.int32),
            pltpu.VMEM((16, EMB), jnp.float32),
            pltpu.VMEM((CHUNK, EMB), jnp.float32),
            pltpu.SemaphoreType.DMA,
            pltpu.SemaphoreType.DMA,
            pltpu.SemaphoreType.DMA,
        ],
    )


def kernel(x):
    batch, seq_len = x.shape
    table = _pos_table(seq_len)
    row0 = _row0_rep()
    return _make_sc_kernel(batch, seq_len)(x, table, row0)
